# baseline (speedup 1.0000x reference)
import jax
import jax.numpy as jnp
from jax import lax
from jax.experimental import pallas as pl
from jax.experimental.pallas import tpu as pltpu
from jax.experimental.pallas import tpu_sc as plsc

BATCH = 16384
EMBED = 64
NC = 2
NS = 16
NW = NC * NS
B_PER_W = BATCH // NW
CHUNK = 128
NCHUNK = B_PER_W // CHUNK


def _sc_kernel(uids_hbm, pids_hbm, user_table, product_table, wb_hbm,
               out_hbm, uidx, pidx, urows, prows, wv, outv, sem):
    wid = lax.axis_index("s") * NC + lax.axis_index("c")
    idrow = wid * NCHUNK

    pltpu.sync_copy(uids_hbm.at[pl.ds(idrow, NCHUNK)], uidx)
    pltpu.sync_copy(pids_hbm.at[pl.ds(idrow, NCHUNK)], pidx)
    pltpu.sync_copy(wb_hbm, wv)

    copies = []
    for j in range(NCHUNK):
        copies.append(pltpu.async_copy(
            user_table.at[uidx.at[j]], urows.at[pl.ds(j * CHUNK, CHUNK)], sem))
        copies.append(pltpu.async_copy(
            product_table.at[pidx.at[j]], prows.at[pl.ds(j * CHUNK, CHUNK)], sem))
    for c in copies:
        c.wait()

    lane = lax.iota(jnp.int32, 16)
    wchunks = [wv[pl.ds(c * 16, 16)] for c in range(EMBED // 16)]
    bias = wv[pl.ds(EMBED, 16)][0]

    def body(g, carry):
        rows = lane + g * 16
        acc = jnp.zeros((16,), jnp.float32)
        for e in range(EMBED):
            col = jnp.full((16,), e, jnp.int32)
            u = plsc.load_gather(urows, [rows, col])
            p = plsc.load_gather(prows, [rows, col])
            acc = acc + u * p * wchunks[e // 16][e % 16]
        outv[pl.ds(g * 16, 16)] = acc + bias
        return carry

    lax.fori_loop(0, B_PER_W // 16, body, 0)

    pltpu.sync_copy(outv, out_hbm.at[pl.ds(wid * B_PER_W, B_PER_W)])


@jax.jit
def kernel(user_ids, product_ids, user_table, product_table, fc_w, fc_b):
    uids2d = user_ids.astype(jnp.int32).reshape(BATCH // CHUNK, CHUNK)
    pids2d = product_ids.astype(jnp.int32).reshape(BATCH // CHUNK, CHUNK)
    wb = jnp.zeros((128,), jnp.float32)
    wb = wb.at[:EMBED].set(fc_w[0]).at[EMBED].set(fc_b[0])

    mesh = plsc.VectorSubcoreMesh(core_axis_name="c", subcore_axis_name="s")
    run = pl.kernel(
        _sc_kernel,
        out_type=jax.ShapeDtypeStruct((BATCH,), jnp.float32),
        mesh=mesh,
        compiler_params=pltpu.CompilerParams(
            use_tc_tiling_on_sc=False, needs_layout_passes=False),
        scratch_types=[
            pltpu.VMEM((NCHUNK, CHUNK), jnp.int32),
            pltpu.VMEM((NCHUNK, CHUNK), jnp.int32),
            pltpu.VMEM((B_PER_W, EMBED), jnp.float32),
            pltpu.VMEM((B_PER_W, EMBED), jnp.float32),
            pltpu.VMEM((128,), jnp.float32),
            pltpu.VMEM((B_PER_W,), jnp.float32),
            pltpu.SemaphoreType.DMA,
        ],
    )
    return run(uids2d, pids2d, user_table, product_table, wb)
